# baseline (speedup 1.0000x reference)
import functools

import jax
import jax.numpy as jnp
from jax import lax
from jax.experimental import pallas as pl
from jax.experimental.pallas import tpu as pltpu
from jax.experimental.pallas import tpu_sc as plsc

NC, NS = 2, 16
NW = NC * NS
CHUNK = 128


def _sc_gather(table, idx3, n_chunks, embed):
    rows_per_w = n_chunks * CHUNK
    total = NW * rows_per_w
    mesh = plsc.VectorSubcoreMesh(core_axis_name="c", subcore_axis_name="s")

    @functools.partial(
        pl.kernel,
        out_type=jax.ShapeDtypeStruct((total, embed), jnp.float32),
        mesh=mesh,
        scratch_types=[
            pltpu.VMEM((n_chunks, CHUNK), jnp.int32),
            pltpu.VMEM((rows_per_w, embed), jnp.float32),
            pltpu.SemaphoreType.DMA,
        ],
        compiler_params=pltpu.CompilerParams(use_tc_tiling_on_sc=False),
    )
    def gather_kernel(table_hbm, idx_hbm, out_hbm, idx_v, rows_v, sem):
        wid = lax.axis_index("s") * NC + lax.axis_index("c")
        pltpu.sync_copy(idx_hbm.at[wid], idx_v)
        copies = [
            pltpu.async_copy(
                table_hbm.at[idx_v.at[j]],
                rows_v.at[pl.ds(j * CHUNK, CHUNK)],
                sem,
            )
            for j in range(n_chunks)
        ]
        for c in copies:
            c.wait()
        pltpu.sync_copy(rows_v, out_hbm.at[pl.ds(wid * rows_per_w, rows_per_w)])

    return gather_kernel(table, idx3)


def kernel(current_node, actionList, table, W, b):
    B = current_node.shape[0]
    embed = table.shape[1]
    last_node = actionList[:, -2]
    idx = jnp.concatenate([current_node, last_node]).astype(jnp.int32)
    n_chunks = (2 * B) // (NW * CHUNK)
    idx3 = idx.reshape(NW, n_chunks, CHUNK)
    gathered = _sc_gather(table, idx3, n_chunks, embed)

    BM = 2048
    grid = B // BM
    w1 = W[:embed]
    w2 = W[embed:]
    b2 = b.reshape(1, embed)

    def proj(cur_ref, last_ref, w1_ref, w2_ref, b_ref, o_ref):
        o_ref[...] = (
            jnp.dot(cur_ref[...], w1_ref[...], preferred_element_type=jnp.float32)
            + jnp.dot(last_ref[...], w2_ref[...], preferred_element_type=jnp.float32)
            + b_ref[...]
        )

    return pl.pallas_call(
        proj,
        grid=(grid,),
        in_specs=[
            pl.BlockSpec((BM, embed), lambda i: (i, 0)),
            pl.BlockSpec((BM, embed), lambda i: (i + grid, 0)),
            pl.BlockSpec((embed, embed), lambda i: (0, 0)),
            pl.BlockSpec((embed, embed), lambda i: (0, 0)),
            pl.BlockSpec((1, embed), lambda i: (0, 0)),
        ],
        out_specs=pl.BlockSpec((BM, embed), lambda i: (i, 0)),
        out_shape=jax.ShapeDtypeStruct((B, embed), jnp.float32),
    )(gathered, gathered, w1, w2, b2)
